# baseline (speedup 1.0000x reference)
import jax
import jax.numpy as jnp
from jax.experimental import pallas as pl


def _xw_kernel(x_ref, w_ref, o_ref):
    o_ref[...] = jnp.dot(
        x_ref[...].astype(jnp.bfloat16),
        w_ref[...],
        preferred_element_type=jnp.float32,
    ).astype(jnp.bfloat16)


def _layer1_kernel(adj_ref, s1_ref, b1_ref, w2_ref, o_ref):
    a = adj_ref[...].astype(jnp.bfloat16)
    h = jnp.dot(a, s1_ref[...], preferred_element_type=jnp.float32)
    h = jnp.maximum(h + b1_ref[...], 0.0)
    o_ref[...] = jnp.dot(
        h.astype(jnp.bfloat16), w2_ref[...], preferred_element_type=jnp.float32
    ).astype(jnp.bfloat16)


def _layer2_kernel(adj_ref, s2_ref, b2_ref, o_ref):
    a = adj_ref[...].astype(jnp.bfloat16)
    z = jnp.dot(a, s2_ref[...], preferred_element_type=jnp.float32)
    o_ref[...] = jax.nn.sigmoid(z + b2_ref[...])


def kernel(x, adj, W1, b1, W2, b2):
    n, nfeat = x.shape
    nhid = W1.shape[1]
    nlabel = W2.shape[1]

    bm1 = 1000 if n % 1000 == 0 else 8
    s1 = pl.pallas_call(
        _xw_kernel,
        grid=(n // bm1,),
        in_specs=[
            pl.BlockSpec((bm1, nfeat), lambda i: (i, 0)),
            pl.BlockSpec((nfeat, nhid), lambda i: (0, 0)),
        ],
        out_specs=pl.BlockSpec((bm1, nhid), lambda i: (i, 0)),
        out_shape=jax.ShapeDtypeStruct((n, nhid), jnp.bfloat16),
    )(x, W1.astype(jnp.bfloat16))

    bm = 200 if n % 200 == 0 else 8
    s2 = pl.pallas_call(
        _layer1_kernel,
        grid=(n // bm,),
        in_specs=[
            pl.BlockSpec((bm, n), lambda i: (i, 0)),
            pl.BlockSpec((n, nhid), lambda i: (0, 0)),
            pl.BlockSpec((1, nhid), lambda i: (0, 0)),
            pl.BlockSpec((nhid, nlabel), lambda i: (0, 0)),
        ],
        out_specs=pl.BlockSpec((bm, nlabel), lambda i: (i, 0)),
        out_shape=jax.ShapeDtypeStruct((n, nlabel), jnp.bfloat16),
    )(adj, s1, b1.reshape(1, nhid), W2.astype(jnp.bfloat16))

    out = pl.pallas_call(
        _layer2_kernel,
        grid=(n // bm,),
        in_specs=[
            pl.BlockSpec((bm, n), lambda i: (i, 0)),
            pl.BlockSpec((n, nlabel), lambda i: (0, 0)),
            pl.BlockSpec((1, nlabel), lambda i: (0, 0)),
        ],
        out_specs=pl.BlockSpec((bm, nlabel), lambda i: (i, 0)),
        out_shape=jax.ShapeDtypeStruct((n, nlabel), jnp.float32),
    )(adj, s2, b2.reshape(1, nlabel))
    return out


# speedup vs baseline: 1.0330x; 1.0330x over previous
import jax
import jax.numpy as jnp
from jax.experimental import pallas as pl


def _xw_kernel(x_ref, w_ref, o_ref):
    o_ref[...] = jnp.dot(
        x_ref[...].astype(jnp.bfloat16),
        w_ref[...],
        preferred_element_type=jnp.float32,
    ).astype(jnp.bfloat16)


def _layer1_kernel(adj_ref, s1_ref, b1_ref, w2_ref, o_ref):
    a = adj_ref[...].astype(jnp.bfloat16)
    h = jnp.dot(a, s1_ref[...], preferred_element_type=jnp.float32)
    h = jnp.maximum(h + b1_ref[...], 0.0)
    o_ref[...] = jnp.dot(
        h.astype(jnp.bfloat16), w2_ref[...], preferred_element_type=jnp.float32
    ).astype(jnp.bfloat16)


def _layer2_kernel(adj_ref, s2_ref, b2_ref, o_ref):
    a = adj_ref[...].astype(jnp.bfloat16)
    z = jnp.dot(a, s2_ref[...], preferred_element_type=jnp.float32)
    o_ref[...] = jax.nn.sigmoid(z + b2_ref[...])


def kernel(x, adj, W1, b1, W2, b2):
    n, nfeat = x.shape
    nhid = W1.shape[1]
    nlabel = W2.shape[1]

    bm1 = 2000 if n % 2000 == 0 else 8
    s1 = pl.pallas_call(
        _xw_kernel,
        grid=(n // bm1,),
        in_specs=[
            pl.BlockSpec((bm1, nfeat), lambda i: (i, 0)),
            pl.BlockSpec((nfeat, nhid), lambda i: (0, 0)),
        ],
        out_specs=pl.BlockSpec((bm1, nhid), lambda i: (i, 0)),
        out_shape=jax.ShapeDtypeStruct((n, nhid), jnp.bfloat16),
    )(x, W1.astype(jnp.bfloat16))

    bm = 400 if n % 400 == 0 else 8
    s2 = pl.pallas_call(
        _layer1_kernel,
        grid=(n // bm,),
        in_specs=[
            pl.BlockSpec((bm, n), lambda i: (i, 0)),
            pl.BlockSpec((n, nhid), lambda i: (0, 0)),
            pl.BlockSpec((1, nhid), lambda i: (0, 0)),
            pl.BlockSpec((nhid, nlabel), lambda i: (0, 0)),
        ],
        out_specs=pl.BlockSpec((bm, nlabel), lambda i: (i, 0)),
        out_shape=jax.ShapeDtypeStruct((n, nlabel), jnp.bfloat16),
    )(adj, s1, b1.reshape(1, nhid), W2.astype(jnp.bfloat16))

    out = pl.pallas_call(
        _layer2_kernel,
        grid=(n // bm,),
        in_specs=[
            pl.BlockSpec((bm, n), lambda i: (i, 0)),
            pl.BlockSpec((n, nlabel), lambda i: (0, 0)),
            pl.BlockSpec((1, nlabel), lambda i: (0, 0)),
        ],
        out_specs=pl.BlockSpec((bm, nlabel), lambda i: (i, 0)),
        out_shape=jax.ShapeDtypeStruct((n, nlabel), jnp.float32),
    )(adj, s2, b2.reshape(1, nlabel))
    return out
